# Initial kernel scaffold; baseline (speedup 1.0000x reference)
import jax
import jax.numpy as jnp
from jax.experimental import pallas as pl


def kernel(dense_input, sparse_input, tables, W_lin, b_lin, W1, b1, g1, bt1, W2, b2, g2, bt2, W_out):
    raise NotImplementedError("write your pallas kernel here")


# speedup vs baseline: 1.2102x; 1.2102x over previous
import functools

import jax
import jax.numpy as jnp
from jax import lax
from jax.experimental import pallas as pl
from jax.experimental.pallas import tpu as pltpu
from jax.experimental.pallas import tpu_sc as plsc

N_DENSE = 13
N_SPARSE = 26
VOCAB = 100000
EMBED = 16
BATCH = 4096
D_IN = N_DENSE + N_SPARSE * EMBED
H1 = 256
H2 = 256
BN_EPS = 1e-3

NC = 2
NS = 16
NW = NC * NS
TOTAL = BATCH * N_SPARSE
ROWS_PER_W = TOTAL // NW
CHUNK = 128
CPW = ROWS_PER_W // CHUNK


def _sc_gather(table_flat, idx3):
    mesh = plsc.VectorSubcoreMesh(core_axis_name="c", subcore_axis_name="s")

    @functools.partial(
        pl.kernel,
        out_type=jax.ShapeDtypeStruct((TOTAL, EMBED), jnp.float32),
        mesh=mesh,
        scratch_types=[
            pltpu.VMEM((CPW, CHUNK), jnp.int32),
            pltpu.VMEM((ROWS_PER_W, EMBED), jnp.float32),
            pltpu.SemaphoreType.DMA,
        ],
        compiler_params=pltpu.CompilerParams(use_tc_tiling_on_sc=False),
    )
    def gather_kernel(table_hbm, idx_hbm, out_hbm, idx_v, rows_v, sem):
        wid = lax.axis_index("s") * NC + lax.axis_index("c")
        pltpu.sync_copy(idx_hbm.at[wid], idx_v)
        copies = []
        for j in range(CPW):
            copies.append(pltpu.async_copy(
                table_hbm.at[idx_v.at[j]],
                rows_v.at[pl.ds(j * CHUNK, CHUNK)],
                sem,
            ))
        for c in copies:
            c.wait()
        pltpu.sync_copy(rows_v, out_hbm.at[pl.ds(wid * ROWS_PER_W, ROWS_PER_W)])

    return gather_kernel(table_flat, idx3)


BLK = 1024


def _dense_body(xd_ref, xe_ref, w1d_ref, w1e_ref, b1_ref, g1_ref, bt1_ref,
                w2_ref, b2_ref, g2_ref, bt2_ref,
                wlind_ref, wline_ref, blin_ref, wout_ref, o_ref):
    inv = 1.0 / (1.0 + BN_EPS) ** 0.5
    xd = xd_ref[...]
    xe = xe_ref[...]
    lin = (jnp.dot(xd, wlind_ref[...], preferred_element_type=jnp.float32)
           + jnp.dot(xe, wline_ref[...], preferred_element_type=jnp.float32)
           + blin_ref[...])
    h = (jnp.dot(xd, w1d_ref[...], preferred_element_type=jnp.float32)
         + jnp.dot(xe, w1e_ref[...], preferred_element_type=jnp.float32)
         + b1_ref[...])
    h = jnp.maximum(h * (g1_ref[...] * inv) + bt1_ref[...], 0.0)
    h = jnp.dot(h, w2_ref[...], preferred_element_type=jnp.float32) + b2_ref[...]
    h = jnp.maximum(h * (g2_ref[...] * inv) + bt2_ref[...], 0.0)
    dnn = jnp.dot(h, wout_ref[...], preferred_element_type=jnp.float32)
    o_ref[...] = jax.nn.sigmoid(lin + dnn)


def _tc_dense(dense_input, emb, W1, b1, g1, bt1, W2, b2, g2, bt2,
              W_lin, b_lin, W_out):
    w1d, w1e = W1[:N_DENSE], W1[N_DENSE:]
    wlind, wline = W_lin[:N_DENSE], W_lin[N_DENSE:]
    row = lambda v: v.reshape(1, -1)
    grid = (BATCH // BLK,)
    full = lambda a: pl.BlockSpec(a.shape, lambda i: (0, 0))
    return pl.pallas_call(
        _dense_body,
        grid=grid,
        in_specs=[
            pl.BlockSpec((BLK, N_DENSE), lambda i: (i, 0)),
            pl.BlockSpec((BLK, N_SPARSE * EMBED), lambda i: (i, 0)),
            full(w1d), full(w1e), full(row(b1)), full(row(g1)), full(row(bt1)),
            full(W2), full(row(b2)), full(row(g2)), full(row(bt2)),
            full(wlind), full(wline), full(row(b_lin)), full(W_out),
        ],
        out_specs=pl.BlockSpec((BLK, 1), lambda i: (i, 0)),
        out_shape=jax.ShapeDtypeStruct((BATCH, 1), jnp.float32),
        compiler_params=pltpu.CompilerParams(
            dimension_semantics=("arbitrary",)),
    )(dense_input, emb, w1d, w1e, row(b1), row(g1), row(bt1),
      W2, row(b2), row(g2), row(bt2), wlind, wline, row(b_lin), W_out)


def kernel(dense_input, sparse_input, tables, W_lin, b_lin,
           W1, b1, g1, bt1, W2, b2, g2, bt2, W_out):
    table_flat = tables.reshape(N_SPARSE * VOCAB, EMBED)
    offs = (jnp.arange(N_SPARSE, dtype=jnp.int32) * VOCAB)[None, :]
    idx3 = (sparse_input + offs).reshape(NW, CPW, CHUNK)
    emb = _sc_gather(table_flat, idx3).reshape(BATCH, N_SPARSE * EMBED)
    return _tc_dense(dense_input, emb, W1, b1, g1, bt1, W2, b2, g2, bt2,
                     W_lin, b_lin, W_out)
